# baseline (speedup 1.0000x reference)
import functools

import jax
import jax.numpy as jnp
from jax import lax
from jax.experimental import pallas as pl
from jax.experimental.pallas import tpu as pltpu
from jax.experimental.pallas import tpu_sc as plsc

_NC = 2
_NS = 16
_NW = _NC * _NS

_D = 64
_SUB = 128
_CHUNK = 1024


@functools.lru_cache(maxsize=None)
def _make_gather(n_total: int):
    assert n_total % _NW == 0
    per_w = n_total // _NW
    assert per_w % _CHUNK == 0
    n_chunks = per_w // _CHUNK
    n_sub = _CHUNK // _SUB

    mesh = plsc.VectorSubcoreMesh(core_axis_name="c", subcore_axis_name="s")

    @functools.partial(
        pl.kernel,
        out_type=jax.ShapeDtypeStruct((n_total, _D), jnp.float32),
        mesh=mesh,
        scratch_types=[
            pltpu.VMEM((per_w,), jnp.int32),
            pltpu.VMEM((_CHUNK, _D), jnp.float32),
            pltpu.SemaphoreType.DMA,
        ],
        compiler_params=pltpu.CompilerParams(use_tc_tiling_on_sc=False),
    )
    def gather(idx_hbm, table_hbm, out_hbm, idx_v, rows_v, sem):
        wid = lax.axis_index("s") * _NC + lax.axis_index("c")
        base = wid * per_w
        pltpu.sync_copy(idx_hbm.at[pl.ds(base, per_w)], idx_v)

        @pl.loop(0, n_chunks)
        def _chunk(ci):
            off = pl.multiple_of(ci * _CHUNK, _CHUNK)
            cps = []
            for j in range(n_sub):
                cps.append(
                    pltpu.async_copy(
                        table_hbm.at[idx_v.at[pl.ds(off + j * _SUB, _SUB)]],
                        rows_v.at[pl.ds(j * _SUB, _SUB)],
                        sem,
                    )
                )
            for cp in cps:
                cp.wait()
            pltpu.sync_copy(rows_v, out_hbm.at[pl.ds(base + off, _CHUNK)])

    return gather


def kernel(input_seq, embedding_weight):
    b, h = input_seq.shape
    n = b * h
    idx = input_seq.reshape(n).astype(jnp.int32)
    out = _make_gather(n)(idx, embedding_weight)
    return out.reshape(b, h, _D)
